# baseline (speedup 1.0000x reference)
import functools

import jax
import jax.numpy as jnp
from jax import lax
from jax.experimental import pallas as pl
from jax.experimental.pallas import tpu as pltpu
from jax.experimental.pallas import tpu_sc as plsc

NC = 2
NS = 16
NW = NC * NS

CHUNK = 128


@functools.cache
def _build_gather(B, V, D):
    assert B % NW == 0
    bpw = B // NW
    assert bpw % CHUNK == 0
    n_chunks = bpw // CHUNK
    assert n_chunks % 2 == 0

    mesh = plsc.VectorSubcoreMesh(core_axis_name="c", subcore_axis_name="s")

    @functools.partial(
        pl.kernel,
        out_type=jax.ShapeDtypeStruct((B, D), jnp.float32),
        mesh=mesh,
        compiler_params=pltpu.CompilerParams(use_tc_tiling_on_sc=False),
        scratch_types=[
            pltpu.VMEM((bpw,), jnp.int32),
            pltpu.VMEM((CHUNK, D), jnp.float32),
            pltpu.VMEM((CHUNK, D), jnp.float32),
            pltpu.SemaphoreType.DMA,
            pltpu.SemaphoreType.DMA,
            pltpu.SemaphoreType.DMA,
            pltpu.SemaphoreType.DMA,
        ],
    )
    def gather_kernel(table, idx_hbm, out_hbm, idx_v, rows0, rows1,
                      gsem0, gsem1, osem0, osem1):
        wid = lax.axis_index("s") * NC + lax.axis_index("c")
        base = wid * bpw
        pltpu.sync_copy(idx_hbm.at[pl.ds(base, bpw)], idx_v)

        rows = (rows0, rows1)
        gsems = (gsem0, gsem1)
        osems = (osem0, osem1)

        def gather_start(c, b):
            off = pl.multiple_of(c * CHUNK, CHUNK)
            pltpu.async_copy(
                table.at[idx_v.at[pl.ds(off, CHUNK)]], rows[b], gsems[b])

        def gather_wait(c, b):
            off = pl.multiple_of(c * CHUNK, CHUNK)
            pltpu.make_async_copy(
                table.at[idx_v.at[pl.ds(off, CHUNK)]], rows[b],
                gsems[b]).wait()

        def out_start(c, b):
            off = pl.multiple_of(c * CHUNK, CHUNK)
            pltpu.async_copy(
                rows[b], out_hbm.at[pl.ds(base + off, CHUNK)], osems[b])

        def out_wait(c, b):
            off = pl.multiple_of(c * CHUNK, CHUNK)
            pltpu.make_async_copy(
                rows[b], out_hbm.at[pl.ds(base + off, CHUNK)],
                osems[b]).wait()

        gather_start(0, 0)
        gather_start(1, 1)

        def step(p, carry):
            for b in range(2):
                c = 2 * p + b
                gather_wait(c, b)
                out_start(c, b)

                @pl.when(c + 2 < n_chunks)
                def _():
                    out_wait(c, b)
                    gather_start(c + 2, b)

            return carry

        lax.fori_loop(0, n_chunks // 2, step, 0)

        for tail in range(max(n_chunks - 2, 0), n_chunks):
            out_wait(tail, tail % 2)

    return gather_kernel


def kernel(x, table):
    B0, S = x.shape
    V, D = table.shape
    B = B0 * S
    flat_idx = x.reshape(B).astype(jnp.int32)
    out = _build_gather(B, V, D)(table, flat_idx)
    return out.reshape(B0, S, D)


# speedup vs baseline: 1.2457x; 1.2457x over previous
import functools

import jax
import jax.numpy as jnp
from jax import lax
from jax.experimental import pallas as pl
from jax.experimental.pallas import tpu as pltpu
from jax.experimental.pallas import tpu_sc as plsc

NC = 2
NS = 16
NW = NC * NS

CHUNK = 128
L = 16


@functools.cache
def _build_gather(B, V, D):
    assert B % NW == 0
    bpw = B // NW
    assert bpw % CHUNK == 0
    n_chunks = bpw // CHUNK
    assert n_chunks % 2 == 0

    mesh = plsc.VectorSubcoreMesh(core_axis_name="c", subcore_axis_name="s")

    @functools.partial(
        pl.kernel,
        out_type=jax.ShapeDtypeStruct((B, D), jnp.float32),
        mesh=mesh,
        compiler_params=pltpu.CompilerParams(use_tc_tiling_on_sc=True),
        scratch_types=[
            pltpu.VMEM((bpw,), jnp.int32),
            pltpu.VMEM((CHUNK, 2 * D), jnp.float32),
            pltpu.VMEM((CHUNK, 2 * D), jnp.float32),
            pltpu.VMEM((CHUNK, D), jnp.float32),
            pltpu.VMEM((CHUNK, D), jnp.float32),
            pltpu.SemaphoreType.DMA,
            pltpu.SemaphoreType.DMA,
            pltpu.SemaphoreType.DMA,
            pltpu.SemaphoreType.DMA,
        ],
    )
    def gather_kernel(tpad, idx_hbm, out_hbm, idx_v, wide0, wide1,
                      pack0, pack1, gsem0, gsem1, osem0, osem1):
        wid = lax.axis_index("s") * NC + lax.axis_index("c")
        base = wid * bpw
        pltpu.sync_copy(idx_hbm.at[pl.ds(base, bpw)], idx_v)

        wides = (wide0, wide1)
        packs = (pack0, pack1)
        gsems = (gsem0, gsem1)
        osems = (osem0, osem1)

        def gather_start(c, b):
            off = pl.multiple_of(c * CHUNK, CHUNK)
            pltpu.async_copy(
                tpad.at[idx_v.at[pl.ds(off, CHUNK)]], wides[b], gsems[b])

        def gather_wait(c, b):
            off = pl.multiple_of(c * CHUNK, CHUNK)
            pltpu.make_async_copy(
                tpad.at[idx_v.at[pl.ds(off, CHUNK)]], wides[b],
                gsems[b]).wait()

        def out_start(c, b):
            off = pl.multiple_of(c * CHUNK, CHUNK)
            pltpu.async_copy(
                packs[b], out_hbm.at[pl.ds(base + off, CHUNK)], osems[b])

        def out_wait(c, b):
            off = pl.multiple_of(c * CHUNK, CHUNK)
            pltpu.make_async_copy(
                packs[b], out_hbm.at[pl.ds(base + off, CHUNK)],
                osems[b]).wait()

        def repack(b):
            wide, packb = wides[b], packs[b]

            def body(r, carry):
                for r8 in range(8):
                    row = r * 8 + r8
                    for j in range(D // L):
                        packb[row, pl.ds(j * L, L)] = (
                            wide[row, pl.ds(j * L, L)])
                return carry

            lax.fori_loop(0, CHUNK // 8, body, 0)

        gather_start(0, 0)
        gather_start(1, 1)

        def step(p, carry):
            for b in range(2):
                c = 2 * p + b
                gather_wait(c, b)

                @pl.when(c >= 2)
                def _():
                    out_wait(c - 2, b)

                repack(b)
                out_start(c, b)

                @pl.when(c + 2 < n_chunks)
                def _():
                    gather_start(c + 2, b)

            return carry

        lax.fori_loop(0, n_chunks // 2, step, 0)

        for tail in range(max(n_chunks - 2, 0), n_chunks):
            out_wait(tail, tail % 2)

    return gather_kernel


def kernel(x, table):
    B0, S = x.shape
    V, D = table.shape
    B = B0 * S
    flat_idx = x.reshape(B).astype(jnp.int32)
    tpad = jnp.pad(table, ((0, 0), (0, D)))
    out = _build_gather(B, V, D)(tpad, flat_idx)
    return out.reshape(B0, S, D)
